# baseline (speedup 1.0000x reference)
import functools

import jax
import jax.numpy as jnp
from jax import lax
from jax.experimental import pallas as pl
from jax.experimental.pallas import tpu as pltpu
from jax.experimental.pallas import tpu_sc as plsc

_B, _T, _D = 2, 2048, 2048
_N = _B * _T
_E, _K = 8, 2
_H = 2 * _D
_NK = _N * _K
_BLK = 256
_G = _NK // _BLK + _E
_R = _G * _BLK
_EPAD = 128


def _router_body(x_ref, wr_ref, i1_ref, i2_ref, w1_ref, w2_ref):
    logits = jnp.dot(x_ref[...].astype(jnp.bfloat16), wr_ref[...],
                     preferred_element_type=jnp.float32)
    col = lax.broadcasted_iota(jnp.int32, logits.shape, 1)
    neg = jnp.float32(-1e30)
    lg1 = jnp.where(col < _E, logits, neg)
    v1 = jnp.max(lg1, axis=1, keepdims=True)
    i1 = jnp.min(jnp.where(lg1 >= v1, col, _EPAD), axis=1)
    lg2 = jnp.where(col == i1[:, None], neg, lg1)
    v2 = jnp.max(lg2, axis=1, keepdims=True)
    i2 = jnp.min(jnp.where(lg2 >= v2, col, _EPAD), axis=1)
    t = jnp.exp(v2 - v1)[:, 0]
    s = 1.0 + t
    i1_ref[...] = i1
    i2_ref[...] = i2
    w1_ref[...] = 1.0 / s
    w2_ref[...] = t / s


def _router(xb, wrp):
    return pl.pallas_call(
        _router_body,
        out_shape=(
            jax.ShapeDtypeStruct((_N,), jnp.int32),
            jax.ShapeDtypeStruct((_N,), jnp.int32),
            jax.ShapeDtypeStruct((_N,), jnp.float32),
            jax.ShapeDtypeStruct((_N,), jnp.float32),
        ),
    )(xb, wrp)


def _sc_gather(table, idx, n_rows, chunk):
    _, ncol = table.shape
    dt = table.dtype
    info = plsc.get_sparse_core_info()
    nw = info.num_cores * info.num_subcores
    rpw = n_rows // nw
    steps = rpw // chunk
    mesh = plsc.VectorSubcoreMesh(core_axis_name="c", subcore_axis_name="s")

    @functools.partial(
        pl.kernel,
        out_type=jax.ShapeDtypeStruct((n_rows, ncol), dt),
        mesh=mesh,
        scratch_types=[
            pltpu.VMEM((2, chunk), jnp.int32),
            pltpu.VMEM((2, chunk, ncol), dt),
            pltpu.SemaphoreType.DMA,
            pltpu.SemaphoreType.DMA,
        ],
    )
    def gather_k(tbl_ref, idx_ref, out_ref, idx_v, rows_v, sem0, sem1):
        wid = lax.axis_index("s") * info.num_cores + lax.axis_index("c")
        base = wid * rpw
        sems = (sem0, sem1)

        def start(i, b):
            pltpu.sync_copy(idx_ref.at[pl.ds(base + i * chunk, chunk)],
                            idx_v.at[b])
            return pltpu.async_copy(tbl_ref.at[idx_v.at[b]], rows_v.at[b],
                                    sems[b])

        prev = start(0, 0)
        for i in range(1, steps):
            b = i % 2
            cur = start(i, b)
            prev.wait()
            pltpu.sync_copy(rows_v.at[1 - b],
                            out_ref.at[pl.ds(base + (i - 1) * chunk, chunk)])
            prev = cur
        prev.wait()
        pltpu.sync_copy(rows_v.at[(steps - 1) % 2],
                        out_ref.at[pl.ds(base + (steps - 1) * chunk, chunk)])

    return gather_k(table, idx)


def _expert_fc_half(xs_half, wfc, be_half, nr_half, goff, h_prev=None):
    htile = _H // 4
    nblk = _G // 2
    with_prev = h_prev is not None

    def body(*refs):
        if with_prev:
            be_ref, nr_ref, hp_ref, xs_ref, wfc_ref, h_ref, wc_ref, cache_ref = refs
            del hp_ref
        else:
            be_ref, nr_ref, xs_ref, wfc_ref, h_ref, wc_ref, cache_ref = refs
        ht = pl.program_id(0)
        g = pl.program_id(1)
        key = be_ref[g] * 4 + ht

        @pl.when(jnp.logical_or(cache_ref[0] != key,
                                jnp.logical_and(g == 0, ht == 0)))
        def _():
            wc_ref[...] = wfc_ref[0].astype(jnp.bfloat16)
            cache_ref[0] = key

        @pl.when(nr_ref[g] > 0)
        def _():
            acc = lax.dot_general(xs_ref[...].astype(jnp.bfloat16), wc_ref[...],
                                  (((1,), (1,)), ((), ())),
                                  preferred_element_type=jnp.float32)
            r = jnp.maximum(acc, 0.0)
            h_ref[...] = (r * r).astype(jnp.bfloat16)

    in_specs = [
        pl.BlockSpec((_BLK, _D), lambda ht, g, be, nr: (g, 0)),
        pl.BlockSpec((1, htile, _D), lambda ht, g, be, nr: (be[g], ht, 0)),
    ]
    args = [be_half, nr_half, xs_half, wfc]
    aliases = {}
    if with_prev:
        in_specs = [pl.BlockSpec(memory_space=pl.ANY)] + in_specs
        args = [be_half, nr_half, h_prev, xs_half, wfc]
        aliases = {2: 0}
    grid_spec = pltpu.PrefetchScalarGridSpec(
        num_scalar_prefetch=2,
        grid=(4, nblk),
        in_specs=in_specs,
        out_specs=pl.BlockSpec((_BLK, htile),
                               lambda ht, g, be, nr: (g + goff, ht)),
        scratch_shapes=[
            pltpu.VMEM((htile, _D), jnp.bfloat16),
            pltpu.SMEM((1,), jnp.int32),
        ],
    )
    return pl.pallas_call(
        body,
        grid_spec=grid_spec,
        out_shape=jax.ShapeDtypeStruct((_R, _H), jnp.bfloat16),
        input_output_aliases=aliases,
        compiler_params=pltpu.CompilerParams(
            dimension_semantics=("arbitrary", "arbitrary")),
    )(*args)


def _expert_proj(h, wproj, w_rep, block_expert, block_nrows):
    dtile = _D // 4

    def body(be_ref, nr_ref, h_ref, wp_ref, w_ref, o_ref, wc_ref, cache_ref):
        dt = pl.program_id(0)
        g = pl.program_id(1)
        key = be_ref[g] * 4 + dt

        @pl.when(jnp.logical_or(cache_ref[0] != key,
                                jnp.logical_and(g == 0, dt == 0)))
        def _():
            wc_ref[...] = wp_ref[0].astype(jnp.bfloat16)
            cache_ref[0] = key

        @pl.when(nr_ref[g] > 0)
        def _():
            acc = lax.dot_general(h_ref[...], wc_ref[...],
                                  (((1,), (1,)), ((), ())),
                                  preferred_element_type=jnp.float32)
            o_ref[...] = acc * w_ref[:, 0:1]

    grid_spec = pltpu.PrefetchScalarGridSpec(
        num_scalar_prefetch=2,
        grid=(4, _G),
        in_specs=[
            pl.BlockSpec((_BLK, _H), lambda dt, g, be, nr: (g, 0)),
            pl.BlockSpec((1, dtile, _H), lambda dt, g, be, nr: (be[g], dt, 0)),
            pl.BlockSpec((_BLK, 128), lambda dt, g, be, nr: (g, 0)),
        ],
        out_specs=pl.BlockSpec((_BLK, dtile), lambda dt, g, be, nr: (g, dt)),
        scratch_shapes=[
            pltpu.VMEM((dtile, _H), jnp.bfloat16),
            pltpu.SMEM((1,), jnp.int32),
        ],
    )
    return pl.pallas_call(
        body,
        grid_spec=grid_spec,
        out_shape=jax.ShapeDtypeStruct((_R, _D), jnp.float32),
        compiler_params=pltpu.CompilerParams(
            dimension_semantics=("arbitrary", "arbitrary")),
    )(block_expert, block_nrows, h, wproj, w_rep)


def _combine(pairs):
    def body(p_ref, o_ref):
        rs = p_ref[...].reshape(tb, _K, _D)
        o_ref[...] = rs[:, 0, :] + rs[:, 1, :]

    tb = 512
    return pl.pallas_call(
        body,
        grid=(_N // tb,),
        in_specs=[pl.BlockSpec((_K * tb, _D), lambda g: (g, 0))],
        out_specs=pl.BlockSpec((tb, _D), lambda g: (g, 0)),
        out_shape=jax.ShapeDtypeStruct((_N, _D), jnp.float32),
        compiler_params=pltpu.CompilerParams(
            dimension_semantics=("parallel",)),
    )(pairs)


def kernel(x, Wr, Wfc, Wproj):
    i32, bf = jnp.int32, jnp.bfloat16
    x2 = x.reshape(_N, _D)
    wrp = jnp.zeros((_D, _EPAD), bf).at[:, :_E].set(Wr.astype(bf).T)

    i1, i2, w1, w2 = _router(x2, wrp)

    e_flat = jnp.stack([i1, i2], axis=1).reshape(-1)
    w_flat = jnp.stack([w1, w2], axis=1).reshape(-1)
    onehot = (e_flat[:, None] == jnp.arange(_E, dtype=i32)[None, :]).astype(i32)
    ranks = jnp.cumsum(onehot, axis=0)
    counts = ranks[-1]
    rank = jnp.take_along_axis(ranks, e_flat[:, None], axis=1)[:, 0] - 1
    padded = ((counts + _BLK - 1) // _BLK) * _BLK
    csp = jnp.cumsum(padded)
    seg_start = csp - padded
    pos_pair = (seg_start[e_flat] + rank).astype(i32)
    src_pair = jnp.zeros((_R,), i32).at[pos_pair].set(
        jnp.arange(_NK, dtype=i32))
    src_token = src_pair // _K
    gstart = jnp.arange(_G, dtype=i32) * _BLK
    block_expert = jnp.minimum(
        jnp.sum((gstart[:, None] >= csp[None, :]).astype(i32), axis=1), _E - 1)
    seg_end_real = (seg_start + counts).astype(i32)
    block_nrows = jnp.clip(seg_end_real[block_expert] - gstart, 0, _BLK)
    block_nrows = block_nrows.astype(i32)
    w_sorted = w_flat[src_pair]
    w_rep = jnp.broadcast_to(w_sorted[:, None], (_R, 128))

    half = _R // 2
    hblk = _G // 2
    xs_a = _sc_gather(x2, src_token[:half], half, 16)
    xs_b = _sc_gather(x2, src_token[half:], half, 16)

    h_a = _expert_fc_half(xs_a, Wfc, block_expert[:hblk], block_nrows[:hblk],
                          0)
    h = _expert_fc_half(xs_b, Wfc, block_expert[hblk:], block_nrows[hblk:],
                        hblk, h_a)
    outs = _expert_proj(h, Wproj, w_rep, block_expert, block_nrows)

    pairs = _sc_gather(outs, pos_pair, _NK, 16)
    out2 = _combine(pairs)
    return out2.reshape(_B, _T, _D)


# speedup vs baseline: 1.0262x; 1.0262x over previous
import functools

import jax
import jax.numpy as jnp
from jax import lax
from jax.experimental import pallas as pl
from jax.experimental.pallas import tpu as pltpu
from jax.experimental.pallas import tpu_sc as plsc

_B, _T, _D = 2, 2048, 2048
_N = _B * _T
_E, _K = 8, 2
_H = 2 * _D
_NK = _N * _K
_BLK = 512
_G = _NK // _BLK + _E
_R = _G * _BLK
_EPAD = 128


def _router_body(x_ref, wr_ref, i1_ref, i2_ref, w1_ref, w2_ref):
    logits = jnp.dot(x_ref[...].astype(jnp.bfloat16), wr_ref[...],
                     preferred_element_type=jnp.float32)
    col = lax.broadcasted_iota(jnp.int32, logits.shape, 1)
    neg = jnp.float32(-1e30)
    lg1 = jnp.where(col < _E, logits, neg)
    v1 = jnp.max(lg1, axis=1, keepdims=True)
    i1 = jnp.min(jnp.where(lg1 >= v1, col, _EPAD), axis=1)
    lg2 = jnp.where(col == i1[:, None], neg, lg1)
    v2 = jnp.max(lg2, axis=1, keepdims=True)
    i2 = jnp.min(jnp.where(lg2 >= v2, col, _EPAD), axis=1)
    t = jnp.exp(v2 - v1)[:, 0]
    s = 1.0 + t
    i1_ref[...] = i1
    i2_ref[...] = i2
    w1_ref[...] = 1.0 / s
    w2_ref[...] = t / s


def _router(xb, wrp):
    return pl.pallas_call(
        _router_body,
        out_shape=(
            jax.ShapeDtypeStruct((_N,), jnp.int32),
            jax.ShapeDtypeStruct((_N,), jnp.int32),
            jax.ShapeDtypeStruct((_N,), jnp.float32),
            jax.ShapeDtypeStruct((_N,), jnp.float32),
        ),
    )(xb, wrp)


def _sc_gather(table, idx, n_rows, chunk):
    _, ncol = table.shape
    dt = table.dtype
    info = plsc.get_sparse_core_info()
    nw = info.num_cores * info.num_subcores
    rpw = n_rows // nw
    steps = rpw // chunk
    mesh = plsc.VectorSubcoreMesh(core_axis_name="c", subcore_axis_name="s")

    @functools.partial(
        pl.kernel,
        out_type=jax.ShapeDtypeStruct((n_rows, ncol), dt),
        mesh=mesh,
        scratch_types=[
            pltpu.VMEM((2, chunk), jnp.int32),
            pltpu.VMEM((2, chunk, ncol), dt),
            pltpu.SemaphoreType.DMA,
            pltpu.SemaphoreType.DMA,
        ],
    )
    def gather_k(tbl_ref, idx_ref, out_ref, idx_v, rows_v, sem0, sem1):
        wid = lax.axis_index("s") * info.num_cores + lax.axis_index("c")
        base = wid * rpw
        sems = (sem0, sem1)

        def start(i, b):
            pltpu.sync_copy(idx_ref.at[pl.ds(base + i * chunk, chunk)],
                            idx_v.at[b])
            return pltpu.async_copy(tbl_ref.at[idx_v.at[b]], rows_v.at[b],
                                    sems[b])

        prev = start(0, 0)
        for i in range(1, steps):
            b = i % 2
            cur = start(i, b)
            prev.wait()
            pltpu.sync_copy(rows_v.at[1 - b],
                            out_ref.at[pl.ds(base + (i - 1) * chunk, chunk)])
            prev = cur
        prev.wait()
        pltpu.sync_copy(rows_v.at[(steps - 1) % 2],
                        out_ref.at[pl.ds(base + (steps - 1) * chunk, chunk)])

    return gather_k(table, idx)


def _expert_fc_half(xs_half, wfc, be_half, nr_half, goff, h_prev=None):
    htile = _H // 2
    nblk = _G // 2
    with_prev = h_prev is not None

    def body(*refs):
        if with_prev:
            be_ref, nr_ref, hp_ref, xs_ref, wfc_ref, h_ref, wc_ref, cache_ref = refs
            del hp_ref
        else:
            be_ref, nr_ref, xs_ref, wfc_ref, h_ref, wc_ref, cache_ref = refs
        ht = pl.program_id(0)
        g = pl.program_id(1)
        key = be_ref[g] * 2 + ht

        @pl.when(jnp.logical_or(cache_ref[0] != key,
                                jnp.logical_and(g == 0, ht == 0)))
        def _():
            wc_ref[...] = wfc_ref[0].astype(jnp.bfloat16)
            cache_ref[0] = key

        @pl.when(nr_ref[g] > 0)
        def _():
            acc = lax.dot_general(xs_ref[...].astype(jnp.bfloat16), wc_ref[...],
                                  (((1,), (1,)), ((), ())),
                                  preferred_element_type=jnp.float32)
            r = jnp.maximum(acc, 0.0)
            h_ref[...] = (r * r).astype(jnp.bfloat16)

    in_specs = [
        pl.BlockSpec((_BLK, _D), lambda ht, g, be, nr: (g, 0)),
        pl.BlockSpec((1, htile, _D), lambda ht, g, be, nr: (be[g], ht, 0)),
    ]
    args = [be_half, nr_half, xs_half, wfc]
    aliases = {}
    if with_prev:
        in_specs = [pl.BlockSpec(memory_space=pl.ANY)] + in_specs
        args = [be_half, nr_half, h_prev, xs_half, wfc]
        aliases = {2: 0}
    grid_spec = pltpu.PrefetchScalarGridSpec(
        num_scalar_prefetch=2,
        grid=(2, nblk),
        in_specs=in_specs,
        out_specs=pl.BlockSpec((_BLK, htile),
                               lambda ht, g, be, nr: (g + goff, ht)),
        scratch_shapes=[
            pltpu.VMEM((htile, _D), jnp.bfloat16),
            pltpu.SMEM((1,), jnp.int32),
        ],
    )
    return pl.pallas_call(
        body,
        grid_spec=grid_spec,
        out_shape=jax.ShapeDtypeStruct((_R, _H), jnp.bfloat16),
        input_output_aliases=aliases,
        compiler_params=pltpu.CompilerParams(
            dimension_semantics=("arbitrary", "arbitrary")),
    )(*args)


def _expert_proj(h, wproj, w_rep, block_expert, block_nrows):
    dtile = _D // 2

    def body(be_ref, nr_ref, h_ref, wp_ref, w_ref, o_ref, wc_ref, cache_ref):
        dt = pl.program_id(0)
        g = pl.program_id(1)
        key = be_ref[g] * 2 + dt

        @pl.when(jnp.logical_or(cache_ref[0] != key,
                                jnp.logical_and(g == 0, dt == 0)))
        def _():
            wc_ref[...] = wp_ref[0].astype(jnp.bfloat16)
            cache_ref[0] = key

        @pl.when(nr_ref[g] > 0)
        def _():
            acc = lax.dot_general(h_ref[...], wc_ref[...],
                                  (((1,), (1,)), ((), ())),
                                  preferred_element_type=jnp.float32)
            o_ref[...] = acc * w_ref[:, 0:1]

    grid_spec = pltpu.PrefetchScalarGridSpec(
        num_scalar_prefetch=2,
        grid=(2, _G),
        in_specs=[
            pl.BlockSpec((_BLK, _H), lambda dt, g, be, nr: (g, 0)),
            pl.BlockSpec((1, dtile, _H), lambda dt, g, be, nr: (be[g], dt, 0)),
            pl.BlockSpec((_BLK, 128), lambda dt, g, be, nr: (g, 0)),
        ],
        out_specs=pl.BlockSpec((_BLK, dtile), lambda dt, g, be, nr: (g, dt)),
        scratch_shapes=[
            pltpu.VMEM((dtile, _H), jnp.bfloat16),
            pltpu.SMEM((1,), jnp.int32),
        ],
    )
    return pl.pallas_call(
        body,
        grid_spec=grid_spec,
        out_shape=jax.ShapeDtypeStruct((_R, _D), jnp.float32),
        compiler_params=pltpu.CompilerParams(
            dimension_semantics=("arbitrary", "arbitrary")),
    )(block_expert, block_nrows, h, wproj, w_rep)


def _combine(pairs):
    def body(p_ref, o_ref):
        rs = p_ref[...].reshape(tb, _K, _D)
        o_ref[...] = rs[:, 0, :] + rs[:, 1, :]

    tb = 512
    return pl.pallas_call(
        body,
        grid=(_N // tb,),
        in_specs=[pl.BlockSpec((_K * tb, _D), lambda g: (g, 0))],
        out_specs=pl.BlockSpec((tb, _D), lambda g: (g, 0)),
        out_shape=jax.ShapeDtypeStruct((_N, _D), jnp.float32),
        compiler_params=pltpu.CompilerParams(
            dimension_semantics=("parallel",)),
    )(pairs)


def kernel(x, Wr, Wfc, Wproj):
    i32, bf = jnp.int32, jnp.bfloat16
    x2 = x.reshape(_N, _D)
    wrp = jnp.zeros((_D, _EPAD), bf).at[:, :_E].set(Wr.astype(bf).T)

    i1, i2, w1, w2 = _router(x2, wrp)

    e_flat = jnp.stack([i1, i2], axis=1).reshape(-1)
    w_flat = jnp.stack([w1, w2], axis=1).reshape(-1)
    onehot = (e_flat[:, None] == jnp.arange(_E, dtype=i32)[None, :]).astype(i32)
    ranks = jnp.cumsum(onehot, axis=0)
    counts = ranks[-1]
    rank = jnp.take_along_axis(ranks, e_flat[:, None], axis=1)[:, 0] - 1
    padded = ((counts + _BLK - 1) // _BLK) * _BLK
    csp = jnp.cumsum(padded)
    seg_start = csp - padded
    pos_pair = (seg_start[e_flat] + rank).astype(i32)
    src_pair = jnp.zeros((_R,), i32).at[pos_pair].set(
        jnp.arange(_NK, dtype=i32))
    src_token = src_pair // _K
    gstart = jnp.arange(_G, dtype=i32) * _BLK
    block_expert = jnp.minimum(
        jnp.sum((gstart[:, None] >= csp[None, :]).astype(i32), axis=1), _E - 1)
    seg_end_real = (seg_start + counts).astype(i32)
    block_nrows = jnp.clip(seg_end_real[block_expert] - gstart, 0, _BLK)
    block_nrows = block_nrows.astype(i32)
    w_sorted = w_flat[src_pair]
    w_rep = jnp.broadcast_to(w_sorted[:, None], (_R, 128))

    half = _R // 2
    hblk = _G // 2
    xs_a = _sc_gather(x2, src_token[:half], half, 16)
    xs_b = _sc_gather(x2, src_token[half:], half, 16)

    h_a = _expert_fc_half(xs_a, Wfc, block_expert[:hblk], block_nrows[:hblk],
                          0)
    h = _expert_fc_half(xs_b, Wfc, block_expert[hblk:], block_nrows[hblk:],
                        hblk, h_a)
    outs = _expert_proj(h, Wproj, w_rep, block_expert, block_nrows)

    pairs = _sc_gather(outs, pos_pair, _NK, 16)
    out2 = _combine(pairs)
    return out2.reshape(_B, _T, _D)


# speedup vs baseline: 1.1476x; 1.1183x over previous
import functools

import jax
import jax.numpy as jnp
from jax import lax
from jax.experimental import pallas as pl
from jax.experimental.pallas import tpu as pltpu
from jax.experimental.pallas import tpu_sc as plsc

_B, _T, _D = 2, 2048, 2048
_N = _B * _T
_E, _K = 8, 2
_H = 2 * _D
_NK = _N * _K
_BLK = 256
_G = _NK // _BLK + _E
_R = _G * _BLK
_EPAD = 128


def _router_body(x_ref, wr_ref, i1_ref, i2_ref, w1_ref, w2_ref):
    logits = jnp.dot(x_ref[...].astype(jnp.bfloat16), wr_ref[...],
                     preferred_element_type=jnp.float32)
    col = lax.broadcasted_iota(jnp.int32, logits.shape, 1)
    neg = jnp.float32(-1e30)
    lg1 = jnp.where(col < _E, logits, neg)
    v1 = jnp.max(lg1, axis=1, keepdims=True)
    i1 = jnp.min(jnp.where(lg1 >= v1, col, _EPAD), axis=1)
    lg2 = jnp.where(col == i1[:, None], neg, lg1)
    v2 = jnp.max(lg2, axis=1, keepdims=True)
    i2 = jnp.min(jnp.where(lg2 >= v2, col, _EPAD), axis=1)
    t = jnp.exp(v2 - v1)[:, 0]
    s = 1.0 + t
    i1_ref[...] = i1
    i2_ref[...] = i2
    w1_ref[...] = 1.0 / s
    w2_ref[...] = t / s


def _router(xb, wrp):
    return pl.pallas_call(
        _router_body,
        out_shape=(
            jax.ShapeDtypeStruct((_N,), jnp.int32),
            jax.ShapeDtypeStruct((_N,), jnp.int32),
            jax.ShapeDtypeStruct((_N,), jnp.float32),
            jax.ShapeDtypeStruct((_N,), jnp.float32),
        ),
    )(xb, wrp)


def _sc_gather(table, idx, n_rows, chunk):
    _, ncol = table.shape
    dt = table.dtype
    info = plsc.get_sparse_core_info()
    nw = info.num_cores * info.num_subcores
    rpw = n_rows // nw
    steps = rpw // chunk
    mesh = plsc.VectorSubcoreMesh(core_axis_name="c", subcore_axis_name="s")

    @functools.partial(
        pl.kernel,
        out_type=jax.ShapeDtypeStruct((n_rows, ncol), dt),
        mesh=mesh,
        scratch_types=[
            pltpu.VMEM((2, chunk), jnp.int32),
            pltpu.VMEM((2, chunk, ncol), dt),
            pltpu.SemaphoreType.DMA,
            pltpu.SemaphoreType.DMA,
        ],
    )
    def gather_k(tbl_ref, idx_ref, out_ref, idx_v, rows_v, sem0, sem1):
        wid = lax.axis_index("s") * info.num_cores + lax.axis_index("c")
        base = wid * rpw
        sems = (sem0, sem1)

        def start(i, b):
            pltpu.sync_copy(idx_ref.at[pl.ds(base + i * chunk, chunk)],
                            idx_v.at[b])
            return pltpu.async_copy(tbl_ref.at[idx_v.at[b]], rows_v.at[b],
                                    sems[b])

        prev = start(0, 0)
        for i in range(1, steps):
            b = i % 2
            cur = start(i, b)
            prev.wait()
            pltpu.sync_copy(rows_v.at[1 - b],
                            out_ref.at[pl.ds(base + (i - 1) * chunk, chunk)])
            prev = cur
        prev.wait()
        pltpu.sync_copy(rows_v.at[(steps - 1) % 2],
                        out_ref.at[pl.ds(base + (steps - 1) * chunk, chunk)])

    return gather_k(table, idx)


def _expert_fc_half(xs_half, wfc, be_half, nr_half, goff, h_prev=None):
    htile = _H // 2
    nblk = _G // 2
    with_prev = h_prev is not None

    def body(*refs):
        if with_prev:
            be_ref, nr_ref, hp_ref, xs_ref, wfc_ref, h_ref, wc_ref, cache_ref = refs
            del hp_ref
        else:
            be_ref, nr_ref, xs_ref, wfc_ref, h_ref, wc_ref, cache_ref = refs
        ht = pl.program_id(0)
        g = pl.program_id(1)
        key = be_ref[g] * 2 + ht

        @pl.when(jnp.logical_or(cache_ref[0] != key,
                                jnp.logical_and(g == 0, ht == 0)))
        def _():
            wc_ref[...] = wfc_ref[0].astype(jnp.bfloat16)
            cache_ref[0] = key

        @pl.when(nr_ref[g] > 0)
        def _():
            acc = lax.dot_general(xs_ref[...].astype(jnp.bfloat16), wc_ref[...],
                                  (((1,), (1,)), ((), ())),
                                  preferred_element_type=jnp.float32)
            r = jnp.maximum(acc, 0.0)
            h_ref[...] = (r * r).astype(jnp.bfloat16)

    in_specs = [
        pl.BlockSpec((_BLK, _D), lambda ht, g, be, nr: (g, 0)),
        pl.BlockSpec((1, htile, _D), lambda ht, g, be, nr: (be[g], ht, 0)),
    ]
    args = [be_half, nr_half, xs_half, wfc]
    aliases = {}
    if with_prev:
        in_specs = [pl.BlockSpec(memory_space=pl.ANY)] + in_specs
        args = [be_half, nr_half, h_prev, xs_half, wfc]
        aliases = {2: 0}
    grid_spec = pltpu.PrefetchScalarGridSpec(
        num_scalar_prefetch=2,
        grid=(2, nblk),
        in_specs=in_specs,
        out_specs=pl.BlockSpec((_BLK, htile),
                               lambda ht, g, be, nr: (g + goff, ht)),
        scratch_shapes=[
            pltpu.VMEM((htile, _D), jnp.bfloat16),
            pltpu.SMEM((1,), jnp.int32),
        ],
    )
    return pl.pallas_call(
        body,
        grid_spec=grid_spec,
        out_shape=jax.ShapeDtypeStruct((_R, _H), jnp.bfloat16),
        input_output_aliases=aliases,
        compiler_params=pltpu.CompilerParams(
            dimension_semantics=("arbitrary", "arbitrary")),
    )(*args)


def _expert_proj(h, wproj, w_rep, block_expert, block_nrows):
    dtile = _D // 2

    def body(be_ref, nr_ref, h_ref, wp_ref, w_ref, o_ref, wc_ref, cache_ref):
        dt = pl.program_id(0)
        g = pl.program_id(1)
        key = be_ref[g] * 2 + dt

        @pl.when(jnp.logical_or(cache_ref[0] != key,
                                jnp.logical_and(g == 0, dt == 0)))
        def _():
            wc_ref[...] = wp_ref[0].astype(jnp.bfloat16)
            cache_ref[0] = key

        @pl.when(nr_ref[g] > 0)
        def _():
            acc = lax.dot_general(h_ref[...], wc_ref[...],
                                  (((1,), (1,)), ((), ())),
                                  preferred_element_type=jnp.float32)
            o_ref[...] = acc * w_ref[:, 0:1]

    grid_spec = pltpu.PrefetchScalarGridSpec(
        num_scalar_prefetch=2,
        grid=(2, _G),
        in_specs=[
            pl.BlockSpec((_BLK, _H), lambda dt, g, be, nr: (g, 0)),
            pl.BlockSpec((1, dtile, _H), lambda dt, g, be, nr: (be[g], dt, 0)),
            pl.BlockSpec((_BLK, 128), lambda dt, g, be, nr: (g, 0)),
        ],
        out_specs=pl.BlockSpec((_BLK, dtile), lambda dt, g, be, nr: (g, dt)),
        scratch_shapes=[
            pltpu.VMEM((dtile, _H), jnp.bfloat16),
            pltpu.SMEM((1,), jnp.int32),
        ],
    )
    return pl.pallas_call(
        body,
        grid_spec=grid_spec,
        out_shape=jax.ShapeDtypeStruct((_R, _D), jnp.float32),
        compiler_params=pltpu.CompilerParams(
            dimension_semantics=("arbitrary", "arbitrary")),
    )(block_expert, block_nrows, h, wproj, w_rep)


def _combine_half(pairs_half, goff, out_prev=None):
    tb = 512
    nblk = _N // 2 // tb
    with_prev = out_prev is not None

    def body(*refs):
        if with_prev:
            op_ref, p_ref, o_ref = refs
            del op_ref
        else:
            p_ref, o_ref = refs
        rs = p_ref[...].reshape(tb, _K, _D)
        o_ref[...] = rs[:, 0, :] + rs[:, 1, :]

    in_specs = [pl.BlockSpec((_K * tb, _D), lambda g: (g, 0))]
    args = [pairs_half]
    aliases = {}
    if with_prev:
        in_specs = [pl.BlockSpec(memory_space=pl.ANY)] + in_specs
        args = [out_prev, pairs_half]
        aliases = {0: 0}
    return pl.pallas_call(
        body,
        grid=(nblk,),
        in_specs=in_specs,
        out_specs=pl.BlockSpec((tb, _D), lambda g: (g + goff, 0)),
        out_shape=jax.ShapeDtypeStruct((_N, _D), jnp.float32),
        input_output_aliases=aliases,
        compiler_params=pltpu.CompilerParams(
            dimension_semantics=("arbitrary",)),
    )(*args)


def kernel(x, Wr, Wfc, Wproj):
    i32, bf = jnp.int32, jnp.bfloat16
    x2 = x.reshape(_N, _D)
    wrp = jnp.zeros((_D, _EPAD), bf).at[:, :_E].set(Wr.astype(bf).T)

    i1, i2, w1, w2 = _router(x2, wrp)

    e_flat = jnp.stack([i1, i2], axis=1).reshape(-1)
    w_flat = jnp.stack([w1, w2], axis=1).reshape(-1)
    onehot = (e_flat[:, None] == jnp.arange(_E, dtype=i32)[None, :]).astype(i32)
    ranks = jnp.cumsum(onehot, axis=0)
    counts = ranks[-1]
    rank = jnp.take_along_axis(ranks, e_flat[:, None], axis=1)[:, 0] - 1
    padded = ((counts + _BLK - 1) // _BLK) * _BLK
    csp = jnp.cumsum(padded)
    seg_start = csp - padded
    pos_pair = (seg_start[e_flat] + rank).astype(i32)
    src_pair = jnp.zeros((_R,), i32).at[pos_pair].set(
        jnp.arange(_NK, dtype=i32))
    src_token = src_pair // _K
    gstart = jnp.arange(_G, dtype=i32) * _BLK
    block_expert = jnp.minimum(
        jnp.sum((gstart[:, None] >= csp[None, :]).astype(i32), axis=1), _E - 1)
    seg_end_real = (seg_start + counts).astype(i32)
    block_nrows = jnp.clip(seg_end_real[block_expert] - gstart, 0, _BLK)
    block_nrows = block_nrows.astype(i32)
    w_sorted = w_flat[src_pair]
    w_rep = jnp.broadcast_to(w_sorted[:, None], (_R, 128))

    half = _R // 2
    hblk = _G // 2
    xs_a = _sc_gather(x2, src_token[:half], half, 16)
    xs_b = _sc_gather(x2, src_token[half:], half, 16)

    h_a = _expert_fc_half(xs_a, Wfc, block_expert[:hblk], block_nrows[:hblk],
                          0)
    h = _expert_fc_half(xs_b, Wfc, block_expert[hblk:], block_nrows[hblk:],
                        hblk, h_a)
    outs = _expert_proj(h, Wproj, w_rep, block_expert, block_nrows)

    phalf = _NK // 2
    pairs_a = _sc_gather(outs, pos_pair[:phalf], phalf, 16)
    pairs_b = _sc_gather(outs, pos_pair[phalf:], phalf, 16)
    out2_a = _combine_half(pairs_a, 0)
    out2 = _combine_half(pairs_b, _N // 2 // 512, out2_a)
    return out2.reshape(_B, _T, _D)
